# Initial kernel scaffold; baseline (speedup 1.0000x reference)
import functools

import jax
import jax.numpy as jnp
from jax import lax
from jax.experimental import pallas as pl
from jax.experimental.pallas import tpu as pltpu
from jax.experimental.pallas import tpu_sc as plsc

N = 10000
E = 160000
D = 256
DH = 128
NC = 2
NS = 16
L = 16
EB = 128
N_PAD = 10240
E_PAD = 163840
EROWS = E_PAD // EB
BPT = E_PAD // NS // EB
RED = N_PAD // (NC * NS)
NSL = N_PAD // NS
BN = 512
RB = N_PAD // BN

_sc_mesh = plsc.VectorSubcoreMesh(
    core_axis_name="c", subcore_axis_name="s", num_cores=NC, num_subcores=NS)


def _rsqrt_newton(x):
    i = plsc.bitcast(x, jnp.int32)
    i = jnp.int32(0x5F3759DF) - lax.shift_right_arithmetic(i, 1)
    y = plsc.bitcast(i, jnp.float32)
    for _ in range(3):
        y = y * (jnp.float32(1.5) - jnp.float32(0.5) * x * y * y)
    return y


@functools.partial(
    pl.kernel,
    out_type=jax.ShapeDtypeStruct((N_PAD,), jnp.float32),
    mesh=_sc_mesh,
    scratch_types=[
        pltpu.VMEM((BPT, EB), jnp.int32),
        pltpu.VMEM((BPT, EB), jnp.float32),
        pltpu.VMEM((N_PAD,), jnp.float32),
        pltpu.VMEM_SHARED((NS, N_PAD), jnp.float32),
        pltpu.VMEM((NS, RED), jnp.float32),
        pltpu.VMEM((RED,), jnp.float32),
    ],
)
def _deg_kernel(dst_hbm, ew_hbm, dinv_hbm, dst_v, ew_v, deg_v, part_sh,
                stage_v, red_v):
    c = lax.axis_index("c")
    s = lax.axis_index("s")
    pltpu.sync_copy(dst_hbm.at[pl.ds(s * BPT, BPT)], dst_v)
    pltpu.sync_copy(ew_hbm.at[pl.ds(s * BPT, BPT)], ew_v)
    zeros = jnp.zeros((L,), jnp.float32)

    @pl.loop(0, N_PAD // L)
    def _zero(i):
        deg_v[pl.ds(i * L, L)] = zeros

    @pl.loop(0, BPT)
    def _acc(b):
        for k in range(EB // L):
            idx = dst_v[b, pl.ds(k * L, L)]
            w = ew_v[b, pl.ds(k * L, L)]
            plsc.addupdate_scatter(deg_v, [idx], w)

    pltpu.sync_copy(deg_v, part_sh.at[s])
    plsc.subcore_barrier()
    rbase = c * (N_PAD // 2) + s * RED
    for t in range(NS):
        pltpu.sync_copy(part_sh.at[t, pl.ds(rbase, RED)], stage_v.at[t])

    @pl.loop(0, RED // L)
    def _red(i):
        acc = stage_v[0, pl.ds(i * L, L)]
        for t in range(1, NS):
            acc = acc + stage_v[t, pl.ds(i * L, L)]
        red_v[pl.ds(i * L, L)] = _rsqrt_newton(acc + jnp.float32(1.0))

    pltpu.sync_copy(red_v, dinv_hbm.at[pl.ds(rbase, RED)])


@functools.partial(
    pl.kernel,
    out_type=jax.ShapeDtypeStruct((NC * N_PAD, DH), jnp.float32),
    mesh=_sc_mesh,
    scratch_types=[
        pltpu.VMEM((BPT, EB), jnp.int32),
        pltpu.VMEM((BPT, EB), jnp.int32),
        pltpu.VMEM((BPT, EB), jnp.float32),
        pltpu.VMEM((EB,), jnp.int32),
        pltpu.VMEM((EB, DH), jnp.float32),
        pltpu.VMEM_SHARED((N_PAD, DH), jnp.float32),
        pltpu.SemaphoreType.DMA,
    ],
)
def _agg_kernel(hs_hbm, src_hbm, dst_hbm, ew_hbm, agg_hbm, src_v, dst_v,
                ew_v, gidx_v, rows_v, acc_sh, sem):
    c = lax.axis_index("c")
    s = lax.axis_index("s")
    rowbase = c * N_PAD
    pltpu.sync_copy(src_hbm.at[pl.ds(s * BPT, BPT)], src_v)
    pltpu.sync_copy(dst_hbm.at[pl.ds(s * BPT, BPT)], dst_v)
    pltpu.sync_copy(ew_hbm.at[pl.ds(s * BPT, BPT)], ew_v)
    zeros = jnp.zeros((L,), jnp.float32)

    @pl.loop(0, EB)
    def _zrow(e):
        for k in range(DH // L):
            rows_v[e, pl.ds(k * L, L)] = zeros

    @pl.loop(0, NSL // EB)
    def _zacc(j):
        pltpu.sync_copy(rows_v, acc_sh.at[pl.ds(s * NSL + j * EB, EB)])

    plsc.subcore_barrier()

    @pl.loop(0, BPT)
    def _edge_batch(b):
        for k in range(EB // L):
            gidx_v[pl.ds(k * L, L)] = src_v[b, pl.ds(k * L, L)] + rowbase
        pltpu.async_copy(hs_hbm.at[gidx_v], rows_v, sem).wait()

        @pl.loop(0, EB)
        def _scale(e):
            w = ew_v[b, e]
            for k in range(DH // L):
                rows_v[e, pl.ds(k * L, L)] = rows_v[e, pl.ds(k * L, L)] * w

        pltpu.sync_copy(rows_v, acc_sh.at[dst_v.at[b]], add=True)

    plsc.subcore_barrier()
    pltpu.sync_copy(acc_sh.at[pl.ds(s * NSL, NSL)],
                    agg_hbm.at[pl.ds(rowbase + s * NSL, NSL)])


def _hs1_body(x_ref, w_ref, dinv_ref, hs_ref):
    h = jnp.dot(x_ref[...], w_ref[...], preferred_element_type=jnp.float32)
    hs_ref[...] = h * dinv_ref[...]


def _hs1_call(x_pad, W1, dinv_col):
    return pl.pallas_call(
        _hs1_body,
        grid=(RB, NC),
        in_specs=[
            pl.BlockSpec((BN, D), lambda i, c: (i, 0)),
            pl.BlockSpec((D, DH), lambda i, c: (0, c)),
            pl.BlockSpec((BN, 1), lambda i, c: (i, 0)),
        ],
        out_specs=pl.BlockSpec((BN, DH), lambda i, c: (c * RB + i, 0)),
        out_shape=jax.ShapeDtypeStruct((NC * N_PAD, DH), jnp.float32),
    )(x_pad, W1, dinv_col)


def _mid_body(agg_ref, hs_ref, dinv_ref, b_ref, w_ref, hs2_ref):
    agg = jnp.concatenate([agg_ref[0], agg_ref[1]], axis=1)
    hs = jnp.concatenate([hs_ref[0], hs_ref[1]], axis=1)
    q = dinv_ref[...] * (agg + hs) + b_ref[...]
    h = jnp.maximum(q, 0.0)
    hs2_ref[...] = dinv_ref[...] * jnp.dot(
        h, w_ref[...], preferred_element_type=jnp.float32)


def _mid_call(agg3, hs3, dinv_col, b1, W2):
    return pl.pallas_call(
        _mid_body,
        grid=(RB, NC),
        in_specs=[
            pl.BlockSpec((NC, BN, DH), lambda i, c: (0, i, 0)),
            pl.BlockSpec((NC, BN, DH), lambda i, c: (0, i, 0)),
            pl.BlockSpec((BN, 1), lambda i, c: (i, 0)),
            pl.BlockSpec((D,), lambda i, c: (0,)),
            pl.BlockSpec((D, DH), lambda i, c: (0, c)),
        ],
        out_specs=pl.BlockSpec((BN, DH), lambda i, c: (c * RB + i, 0)),
        out_shape=jax.ShapeDtypeStruct((NC * N_PAD, DH), jnp.float32),
    )(agg3, hs3, dinv_col, b1, W2)


def _fin_body(agg_ref, hs_ref, dinv_ref, b_ref, out_ref):
    agg = jnp.concatenate([agg_ref[0], agg_ref[1]], axis=1)
    hs = jnp.concatenate([hs_ref[0], hs_ref[1]], axis=1)
    out_ref[...] = dinv_ref[...] * (agg + hs) + b_ref[...]


_FBN = 400


def _fin_call(agg3, hs3, dinv_col, b2):
    return pl.pallas_call(
        _fin_body,
        grid=(N // _FBN,),
        in_specs=[
            pl.BlockSpec((NC, _FBN, DH), lambda i: (0, i, 0)),
            pl.BlockSpec((NC, _FBN, DH), lambda i: (0, i, 0)),
            pl.BlockSpec((_FBN, 1), lambda i: (i, 0)),
            pl.BlockSpec((D,), lambda i: (0,)),
        ],
        out_specs=pl.BlockSpec((_FBN, D), lambda i: (i, 0)),
        out_shape=jax.ShapeDtypeStruct((N, D), jnp.float32),
    )(agg3, hs3, dinv_col, b2)


def kernel(x, edge_index, edge_weight, W1, b1, W2, b2):
    src = jnp.pad(edge_index[0], (0, E_PAD - E)).reshape(EROWS, EB)
    dst = jnp.pad(edge_index[1], (0, E_PAD - E)).reshape(EROWS, EB)
    ew = jnp.pad(edge_weight, (0, E_PAD - E)).reshape(EROWS, EB)
    x_pad = jnp.pad(x, ((0, N_PAD - N), (0, 0)))

    dinv = _deg_kernel(dst, ew)
    dinv_col = dinv.reshape(N_PAD, 1)

    hs1 = _hs1_call(x_pad, W1, dinv_col)
    agg1 = _agg_kernel(hs1, src, dst, ew)

    hs2 = _mid_call(agg1.reshape(NC, N_PAD, DH), hs1.reshape(NC, N_PAD, DH),
                    dinv_col, b1, W2)
    agg2 = _agg_kernel(hs2, src, dst, ew)

    out = _fin_call(agg2.reshape(NC, N_PAD, DH), hs2.reshape(NC, N_PAD, DH),
                    dinv_col, b2)
    return out


# speedup vs baseline: 6.0449x; 6.0449x over previous
import functools

import jax
import jax.numpy as jnp
from jax import lax
from jax.experimental import pallas as pl
from jax.experimental.pallas import tpu as pltpu
from jax.experimental.pallas import tpu_sc as plsc

N = 10000
E = 160000
D = 256
DH = 128
NC = 2
NS = 16
L = 16
EB = 128
N_PAD = 10240
E_PAD = 163840
EROWS = E_PAD // EB
BPT = E_PAD // NS // EB
RED = N_PAD // (NC * NS)
NSL = N_PAD // NS
BN = 512
RB = N_PAD // BN

_sc_mesh = plsc.VectorSubcoreMesh(
    core_axis_name="c", subcore_axis_name="s", num_cores=NC, num_subcores=NS)


def _rsqrt_newton(x):
    i = plsc.bitcast(x, jnp.int32)
    i = jnp.int32(0x5F3759DF) - lax.shift_right_arithmetic(i, 1)
    y = plsc.bitcast(i, jnp.float32)
    for _ in range(3):
        y = y * (jnp.float32(1.5) - jnp.float32(0.5) * x * y * y)
    return y


@functools.partial(
    pl.kernel,
    out_type=jax.ShapeDtypeStruct((N_PAD,), jnp.float32),
    mesh=_sc_mesh,
    compiler_params=pltpu.CompilerParams(needs_layout_passes=False),
    scratch_types=[
        pltpu.VMEM((BPT, EB), jnp.int32),
        pltpu.VMEM((BPT, EB), jnp.float32),
        pltpu.VMEM((N_PAD,), jnp.float32),
        pltpu.VMEM_SHARED((NS * N_PAD,), jnp.float32),
        pltpu.VMEM((NS * RED,), jnp.float32),
        pltpu.VMEM((RED,), jnp.float32),
    ],
)
def _deg_kernel(dst_hbm, ew_hbm, dinv_hbm, dst_v, ew_v, deg_v, part_sh,
                stage_v, red_v):
    c = lax.axis_index("c")
    s = lax.axis_index("s")
    pltpu.sync_copy(dst_hbm.at[pl.ds(s * BPT, BPT)], dst_v)
    pltpu.sync_copy(ew_hbm.at[pl.ds(s * BPT, BPT)], ew_v)
    zeros = jnp.zeros((L,), jnp.float32)

    @pl.loop(0, N_PAD // L)
    def _zero(i):
        deg_v[pl.ds(i * L, L)] = zeros

    @pl.loop(0, BPT)
    def _acc(b):
        for k in range(EB // L):
            idx = dst_v[b, pl.ds(k * L, L)]
            w = ew_v[b, pl.ds(k * L, L)]
            plsc.addupdate_scatter(deg_v, [idx], w)

    pltpu.sync_copy(deg_v, part_sh.at[pl.ds(s * N_PAD, N_PAD)])
    plsc.subcore_barrier()
    rbase = c * (N_PAD // 2) + s * RED
    for t in range(NS):
        pltpu.sync_copy(part_sh.at[pl.ds(t * N_PAD + rbase, RED)],
                        stage_v.at[pl.ds(t * RED, RED)])

    @pl.loop(0, RED // L)
    def _red(i):
        acc = stage_v[pl.ds(i * L, L)]
        for t in range(1, NS):
            acc = acc + stage_v[pl.ds(t * RED + i * L, L)]
        red_v[pl.ds(i * L, L)] = _rsqrt_newton(acc + jnp.float32(1.0))

    pltpu.sync_copy(red_v, dinv_hbm.at[pl.ds(rbase, RED)])


@functools.partial(
    pl.kernel,
    out_type=jax.ShapeDtypeStruct((NC * N_PAD, DH), jnp.float32),
    mesh=_sc_mesh,
    compiler_params=pltpu.CompilerParams(needs_layout_passes=False),
    scratch_types=[
        pltpu.VMEM((BPT, EB), jnp.int32),
        pltpu.VMEM((BPT, EB), jnp.int32),
        pltpu.VMEM((BPT, EB), jnp.float32),
        pltpu.VMEM((EB,), jnp.int32),
        pltpu.VMEM((EB, DH), jnp.float32),
        pltpu.VMEM_SHARED((N_PAD, DH), jnp.float32),
        pltpu.SemaphoreType.DMA,
    ],
)
def _agg_kernel(hs_hbm, src_hbm, dst_hbm, ew_hbm, agg_hbm, src_v, dst_v,
                ew_v, gidx_v, rows_v, acc_sh, sem):
    c = lax.axis_index("c")
    s = lax.axis_index("s")
    rowbase = c * N_PAD
    pltpu.sync_copy(src_hbm.at[pl.ds(s * BPT, BPT)], src_v)
    pltpu.sync_copy(dst_hbm.at[pl.ds(s * BPT, BPT)], dst_v)
    pltpu.sync_copy(ew_hbm.at[pl.ds(s * BPT, BPT)], ew_v)
    zeros = jnp.zeros((L,), jnp.float32)

    @pl.loop(0, EB)
    def _zrow(e):
        for k in range(DH // L):
            rows_v[e, pl.ds(k * L, L)] = zeros

    @pl.loop(0, NSL // EB)
    def _zacc(j):
        pltpu.sync_copy(rows_v, acc_sh.at[pl.ds(s * NSL + j * EB, EB)])

    plsc.subcore_barrier()

    @pl.loop(0, BPT)
    def _edge_batch(b):
        for k in range(EB // L):
            gidx_v[pl.ds(k * L, L)] = src_v[b, pl.ds(k * L, L)] + rowbase
        pltpu.async_copy(hs_hbm.at[gidx_v], rows_v, sem).wait()

        @pl.loop(0, EB // L)
        def _scale(g):
            w16 = ew_v[b, pl.ds(g * L, L)]
            for j in range(L):
                w = w16[j]
                e = g * L + j
                for k in range(DH // L):
                    rows_v[e, pl.ds(k * L, L)] = (
                        rows_v[e, pl.ds(k * L, L)] * w)

        pltpu.sync_copy(rows_v, acc_sh.at[dst_v.at[b]], add=True)

    plsc.subcore_barrier()
    pltpu.sync_copy(acc_sh.at[pl.ds(s * NSL, NSL)],
                    agg_hbm.at[pl.ds(rowbase + s * NSL, NSL)])


def _hs1_body(x_ref, w_ref, dinv_ref, hs_ref):
    h = jnp.dot(x_ref[...], w_ref[...], preferred_element_type=jnp.float32)
    hs_ref[...] = h * dinv_ref[...]


def _hs1_call(x_pad, W1, dinv_col):
    return pl.pallas_call(
        _hs1_body,
        grid=(RB, NC),
        in_specs=[
            pl.BlockSpec((BN, D), lambda i, c: (i, 0)),
            pl.BlockSpec((D, DH), lambda i, c: (0, c)),
            pl.BlockSpec((BN, 1), lambda i, c: (i, 0)),
        ],
        out_specs=pl.BlockSpec((BN, DH), lambda i, c: (c * RB + i, 0)),
        out_shape=jax.ShapeDtypeStruct((NC * N_PAD, DH), jnp.float32),
    )(x_pad, W1, dinv_col)


def _mid_body(agg_ref, hs_ref, dinv_ref, b_ref, w_ref, hs2_ref):
    agg = jnp.concatenate([agg_ref[0], agg_ref[1]], axis=1)
    hs = jnp.concatenate([hs_ref[0], hs_ref[1]], axis=1)
    q = dinv_ref[...] * (agg + hs) + b_ref[...]
    h = jnp.maximum(q, 0.0)
    hs2_ref[...] = dinv_ref[...] * jnp.dot(
        h, w_ref[...], preferred_element_type=jnp.float32)


def _mid_call(agg3, hs3, dinv_col, b1, W2):
    return pl.pallas_call(
        _mid_body,
        grid=(RB, NC),
        in_specs=[
            pl.BlockSpec((NC, BN, DH), lambda i, c: (0, i, 0)),
            pl.BlockSpec((NC, BN, DH), lambda i, c: (0, i, 0)),
            pl.BlockSpec((BN, 1), lambda i, c: (i, 0)),
            pl.BlockSpec((D,), lambda i, c: (0,)),
            pl.BlockSpec((D, DH), lambda i, c: (0, c)),
        ],
        out_specs=pl.BlockSpec((BN, DH), lambda i, c: (c * RB + i, 0)),
        out_shape=jax.ShapeDtypeStruct((NC * N_PAD, DH), jnp.float32),
    )(agg3, hs3, dinv_col, b1, W2)


def _fin_body(agg_ref, hs_ref, dinv_ref, b_ref, out_ref):
    agg = jnp.concatenate([agg_ref[0], agg_ref[1]], axis=1)
    hs = jnp.concatenate([hs_ref[0], hs_ref[1]], axis=1)
    out_ref[...] = dinv_ref[...] * (agg + hs) + b_ref[...]


_FBN = 400


def _fin_call(agg3, hs3, dinv_col, b2):
    return pl.pallas_call(
        _fin_body,
        grid=(N // _FBN,),
        in_specs=[
            pl.BlockSpec((NC, _FBN, DH), lambda i: (0, i, 0)),
            pl.BlockSpec((NC, _FBN, DH), lambda i: (0, i, 0)),
            pl.BlockSpec((_FBN, 1), lambda i: (i, 0)),
            pl.BlockSpec((D,), lambda i: (0,)),
        ],
        out_specs=pl.BlockSpec((_FBN, D), lambda i: (i, 0)),
        out_shape=jax.ShapeDtypeStruct((N, D), jnp.float32),
    )(agg3, hs3, dinv_col, b2)


def kernel(x, edge_index, edge_weight, W1, b1, W2, b2):
    src = jnp.pad(edge_index[0], (0, E_PAD - E)).reshape(EROWS, EB)
    dst = jnp.pad(edge_index[1], (0, E_PAD - E)).reshape(EROWS, EB)
    ew = jnp.pad(edge_weight, (0, E_PAD - E)).reshape(EROWS, EB)
    x_pad = jnp.pad(x, ((0, N_PAD - N), (0, 0)))

    dinv = _deg_kernel(dst, ew)
    dinv_col = dinv.reshape(N_PAD, 1)

    hs1 = _hs1_call(x_pad, W1, dinv_col)
    agg1 = _agg_kernel(hs1, src, dst, ew)

    hs2 = _mid_call(agg1.reshape(NC, N_PAD, DH), hs1.reshape(NC, N_PAD, DH),
                    dinv_col, b1, W2)
    agg2 = _agg_kernel(hs2, src, dst, ew)

    out = _fin_call(agg2.reshape(NC, N_PAD, DH), hs2.reshape(NC, N_PAD, DH),
                    dinv_col, b2)
    return out


# speedup vs baseline: 6.5163x; 1.0780x over previous
import functools

import jax
import jax.numpy as jnp
from jax import lax
from jax.experimental import pallas as pl
from jax.experimental.pallas import tpu as pltpu
from jax.experimental.pallas import tpu_sc as plsc

N = 10000
E = 160000
D = 256
DH = 128
NC = 2
NS = 16
L = 16
EB = 64
N_PAD = 10240
E_PAD = 163840
EROWS = E_PAD // EB
BPT = E_PAD // NS // EB
RED = N_PAD // (NC * NS)
NSL = N_PAD // NS
BN = 512
RB = N_PAD // BN

_sc_mesh = plsc.VectorSubcoreMesh(
    core_axis_name="c", subcore_axis_name="s", num_cores=NC, num_subcores=NS)


def _rsqrt_newton(x):
    i = plsc.bitcast(x, jnp.int32)
    i = jnp.int32(0x5F3759DF) - lax.shift_right_arithmetic(i, 1)
    y = plsc.bitcast(i, jnp.float32)
    for _ in range(3):
        y = y * (jnp.float32(1.5) - jnp.float32(0.5) * x * y * y)
    return y


@functools.partial(
    pl.kernel,
    out_type=jax.ShapeDtypeStruct((N_PAD,), jnp.float32),
    mesh=_sc_mesh,
    compiler_params=pltpu.CompilerParams(needs_layout_passes=False),
    scratch_types=[
        pltpu.VMEM((BPT, EB), jnp.int32),
        pltpu.VMEM((BPT, EB), jnp.float32),
        pltpu.VMEM((N_PAD,), jnp.float32),
        pltpu.VMEM_SHARED((NS * N_PAD,), jnp.float32),
        pltpu.VMEM((NS * RED,), jnp.float32),
        pltpu.VMEM((RED,), jnp.float32),
    ],
)
def _deg_kernel(dst_hbm, ew_hbm, dinv_hbm, dst_v, ew_v, deg_v, part_sh,
                stage_v, red_v):
    c = lax.axis_index("c")
    s = lax.axis_index("s")
    pltpu.sync_copy(dst_hbm.at[pl.ds(s * BPT, BPT)], dst_v)
    pltpu.sync_copy(ew_hbm.at[pl.ds(s * BPT, BPT)], ew_v)
    zeros = jnp.zeros((L,), jnp.float32)

    @pl.loop(0, N_PAD // L)
    def _zero(i):
        deg_v[pl.ds(i * L, L)] = zeros

    @pl.loop(0, BPT)
    def _acc(b):
        for k in range(EB // L):
            idx = dst_v[b, pl.ds(k * L, L)]
            w = ew_v[b, pl.ds(k * L, L)]
            plsc.addupdate_scatter(deg_v, [idx], w)

    pltpu.sync_copy(deg_v, part_sh.at[pl.ds(s * N_PAD, N_PAD)])
    plsc.subcore_barrier()
    rbase = c * (N_PAD // 2) + s * RED
    for t in range(NS):
        pltpu.sync_copy(part_sh.at[pl.ds(t * N_PAD + rbase, RED)],
                        stage_v.at[pl.ds(t * RED, RED)])

    @pl.loop(0, RED // L)
    def _red(i):
        acc = stage_v[pl.ds(i * L, L)]
        for t in range(1, NS):
            acc = acc + stage_v[pl.ds(t * RED + i * L, L)]
        red_v[pl.ds(i * L, L)] = _rsqrt_newton(acc + jnp.float32(1.0))

    pltpu.sync_copy(red_v, dinv_hbm.at[pl.ds(rbase, RED)])


@functools.partial(
    pl.kernel,
    out_type=jax.ShapeDtypeStruct((NC * N_PAD, DH), jnp.float32),
    mesh=_sc_mesh,
    compiler_params=pltpu.CompilerParams(needs_layout_passes=False),
    scratch_types=[
        pltpu.VMEM((3, EB), jnp.int32),
        pltpu.VMEM((3, EB), jnp.int32),
        pltpu.VMEM((3, EB), jnp.int32),
        pltpu.VMEM((3, EB), jnp.int32),
        pltpu.VMEM((EB,), jnp.int32),
        pltpu.VMEM((EB,), jnp.int32),
        pltpu.VMEM((EB, DH), jnp.float32),
        pltpu.VMEM((EB, DH), jnp.float32),
        pltpu.VMEM_SHARED((N_PAD, DH), jnp.float32),
        pltpu.SemaphoreType.DMA,
        pltpu.SemaphoreType.DMA,
        pltpu.SemaphoreType.DMA,
        pltpu.SemaphoreType.DMA,
        pltpu.SemaphoreType.DMA,
        pltpu.SemaphoreType.DMA,
        pltpu.SemaphoreType.DMA,
        pltpu.SemaphoreType.DMA,
    ],
)
def _agg_kernel(hs_hbm, ed_hbm, agg_hbm, edA, edB, edC, edD,
                gidx0_v, gidx1_v, rows0_v, rows1_v, acc_sh,
                g0, g1, s0, s1, eA, eB, eC, eD):
    c = lax.axis_index("c")
    s = lax.axis_index("s")
    rowbase = c * N_PAD
    ebase = s * BPT
    zeros = jnp.zeros((L,), jnp.float32)

    @pl.loop(0, EB)
    def _zrow(e):
        for k in range(DH // L):
            rows0_v[e, pl.ds(k * L, L)] = zeros

    @pl.loop(0, NSL // EB)
    def _zacc(j):
        pltpu.sync_copy(rows0_v, acc_sh.at[pl.ds(s * NSL + j * EB, EB)])

    plsc.subcore_barrier()

    def _wrap(b):
        return jnp.where(b >= BPT, b - BPT, b)

    def _eload(b, ed, esem):
        pltpu.async_copy(ed_hbm.at[ebase + _wrap(b)], ed, esem)

    def _ewait(b, ed, esem):
        pltpu.make_async_copy(ed_hbm.at[ebase + _wrap(b)], ed, esem).wait()

    def _mk_gidx(ed, gidx_v):
        for k in range(EB // L):
            gidx_v[pl.ds(k * L, L)] = ed[0, pl.ds(k * L, L)] + rowbase

    def _scale(ed, rows_v):
        @pl.loop(0, EB // L)
        def _sc(g):
            w16 = plsc.bitcast(ed[2, pl.ds(g * L, L)], jnp.float32)
            for j in range(L):
                w = w16[j]
                e = g * L + j
                for k in range(DH // L):
                    rows_v[e, pl.ds(k * L, L)] = (
                        rows_v[e, pl.ds(k * L, L)] * w)

    def _gather(gidx_v, rows_v, gsem):
        pltpu.async_copy(hs_hbm.at[gidx_v], rows_v, gsem)

    def _gwait(gidx_v, rows_v, gsem):
        pltpu.make_async_copy(hs_hbm.at[gidx_v], rows_v, gsem).wait()

    def _scat(ed, rows_v, ssem):
        pltpu.async_copy(rows_v, acc_sh.at[ed.at[1]], ssem, add=True)

    def _swait(ed, rows_v, ssem):
        pltpu.make_async_copy(rows_v, acc_sh.at[ed.at[1]], ssem).wait()

    pltpu.sync_copy(ed_hbm.at[ebase], edA)
    pltpu.sync_copy(ed_hbm.at[ebase + 1], edB)
    _eload(2, edC, eC)
    _eload(3, edD, eD)
    _mk_gidx(edA, gidx0_v)
    _gather(gidx0_v, rows0_v, g0)
    _mk_gidx(edB, gidx1_v)
    _gather(gidx1_v, rows1_v, g1)

    @pl.loop(0, BPT // 4)
    def _edge_batches(kk):
        b = kk * 4
        _gwait(gidx0_v, rows0_v, g0)
        _scale(edA, rows0_v)
        _scat(edA, rows0_v, s0)
        _gwait(gidx1_v, rows1_v, g1)
        _scale(edB, rows1_v)
        _scat(edB, rows1_v, s1)
        _ewait(b + 2, edC, eC)
        _mk_gidx(edC, gidx0_v)
        _swait(edA, rows0_v, s0)
        _gather(gidx0_v, rows0_v, g0)
        _eload(b + 4, edA, eA)
        _ewait(b + 3, edD, eD)
        _mk_gidx(edD, gidx1_v)
        _swait(edB, rows1_v, s1)
        _gather(gidx1_v, rows1_v, g1)
        _eload(b + 5, edB, eB)
        _gwait(gidx0_v, rows0_v, g0)
        _scale(edC, rows0_v)
        _scat(edC, rows0_v, s0)
        _gwait(gidx1_v, rows1_v, g1)
        _scale(edD, rows1_v)
        _scat(edD, rows1_v, s1)
        _ewait(b + 4, edA, eA)
        _mk_gidx(edA, gidx0_v)
        _swait(edC, rows0_v, s0)
        _gather(gidx0_v, rows0_v, g0)
        _eload(b + 6, edC, eC)
        _ewait(b + 5, edB, eB)
        _mk_gidx(edB, gidx1_v)
        _swait(edD, rows1_v, s1)
        _gather(gidx1_v, rows1_v, g1)
        _eload(b + 7, edD, eD)

    _gwait(gidx0_v, rows0_v, g0)
    _gwait(gidx1_v, rows1_v, g1)
    _ewait(2, edC, eC)
    _ewait(3, edD, eD)

    plsc.subcore_barrier()
    pltpu.sync_copy(acc_sh.at[pl.ds(s * NSL, NSL)],
                    agg_hbm.at[pl.ds(rowbase + s * NSL, NSL)])


def _hs1_body(x_ref, w_ref, dinv_ref, hs_ref):
    h = jnp.dot(x_ref[...], w_ref[...], preferred_element_type=jnp.float32)
    hs_ref[...] = h * dinv_ref[...]


def _hs1_call(x_pad, W1, dinv_col):
    return pl.pallas_call(
        _hs1_body,
        grid=(RB, NC),
        in_specs=[
            pl.BlockSpec((BN, D), lambda i, c: (i, 0)),
            pl.BlockSpec((D, DH), lambda i, c: (0, c)),
            pl.BlockSpec((BN, 1), lambda i, c: (i, 0)),
        ],
        out_specs=pl.BlockSpec((BN, DH), lambda i, c: (c * RB + i, 0)),
        out_shape=jax.ShapeDtypeStruct((NC * N_PAD, DH), jnp.float32),
    )(x_pad, W1, dinv_col)


def _mid_body(agg_ref, hs_ref, dinv_ref, b_ref, w_ref, hs2_ref):
    agg = jnp.concatenate([agg_ref[0], agg_ref[1]], axis=1)
    hs = jnp.concatenate([hs_ref[0], hs_ref[1]], axis=1)
    q = dinv_ref[...] * (agg + hs) + b_ref[...]
    h = jnp.maximum(q, 0.0)
    hs2_ref[...] = dinv_ref[...] * jnp.dot(
        h, w_ref[...], preferred_element_type=jnp.float32)


def _mid_call(agg3, hs3, dinv_col, b1, W2):
    return pl.pallas_call(
        _mid_body,
        grid=(RB, NC),
        in_specs=[
            pl.BlockSpec((NC, BN, DH), lambda i, c: (0, i, 0)),
            pl.BlockSpec((NC, BN, DH), lambda i, c: (0, i, 0)),
            pl.BlockSpec((BN, 1), lambda i, c: (i, 0)),
            pl.BlockSpec((D,), lambda i, c: (0,)),
            pl.BlockSpec((D, DH), lambda i, c: (0, c)),
        ],
        out_specs=pl.BlockSpec((BN, DH), lambda i, c: (c * RB + i, 0)),
        out_shape=jax.ShapeDtypeStruct((NC * N_PAD, DH), jnp.float32),
    )(agg3, hs3, dinv_col, b1, W2)


def _fin_body(agg_ref, hs_ref, dinv_ref, b_ref, out_ref):
    agg = jnp.concatenate([agg_ref[0], agg_ref[1]], axis=1)
    hs = jnp.concatenate([hs_ref[0], hs_ref[1]], axis=1)
    out_ref[...] = dinv_ref[...] * (agg + hs) + b_ref[...]


_FBN = 400


def _fin_call(agg3, hs3, dinv_col, b2):
    return pl.pallas_call(
        _fin_body,
        grid=(N // _FBN,),
        in_specs=[
            pl.BlockSpec((NC, _FBN, DH), lambda i: (0, i, 0)),
            pl.BlockSpec((NC, _FBN, DH), lambda i: (0, i, 0)),
            pl.BlockSpec((_FBN, 1), lambda i: (i, 0)),
            pl.BlockSpec((D,), lambda i: (0,)),
        ],
        out_specs=pl.BlockSpec((_FBN, D), lambda i: (i, 0)),
        out_shape=jax.ShapeDtypeStruct((N, D), jnp.float32),
    )(agg3, hs3, dinv_col, b2)


def kernel(x, edge_index, edge_weight, W1, b1, W2, b2):
    src = jnp.pad(edge_index[0], (0, E_PAD - E)).reshape(EROWS, EB)
    dst = jnp.pad(edge_index[1], (0, E_PAD - E)).reshape(EROWS, EB)
    ew = jnp.pad(edge_weight, (0, E_PAD - E)).reshape(EROWS, EB)
    ew_bits = lax.bitcast_convert_type(ew, jnp.int32)
    ed = jnp.stack([src, dst, ew_bits], axis=1)
    x_pad = jnp.pad(x, ((0, N_PAD - N), (0, 0)))

    dinv = _deg_kernel(dst, ew)
    dinv_col = dinv.reshape(N_PAD, 1)

    hs1 = _hs1_call(x_pad, W1, dinv_col)
    agg1 = _agg_kernel(hs1, ed)

    hs2 = _mid_call(agg1.reshape(NC, N_PAD, DH), hs1.reshape(NC, N_PAD, DH),
                    dinv_col, b1, W2)
    agg2 = _agg_kernel(hs2, ed)

    out = _fin_call(agg2.reshape(NC, N_PAD, DH), hs2.reshape(NC, N_PAD, DH),
                    dinv_col, b2)
    return out
